# baseline (speedup 1.0000x reference)
import jax
import jax.numpy as jnp
from jax import lax
from jax.experimental import pallas as pl
from jax.experimental.pallas import tpu as pltpu
from jax.experimental.pallas import tpu_sc as plsc

_INFO = plsc.get_sparse_core_info()
_NC = _INFO.num_cores
_NS = _INFO.num_subcores
_NW = _NC * _NS


def _sc_copy_body(src_hbm, out_hbm, buf0, buf1, si0, si1, so0, so1):
    wid = lax.axis_index("s") * _NC + lax.axis_index("c")
    n = src_hbm.shape[0] // _NW
    chunk = buf0.shape[0]
    nchunks = n // chunk
    base = wid * n
    bufs = (buf0, buf1)
    sin = (si0, si1)
    sout = (so0, so1)

    def in_copy(i):
        return pltpu.make_async_copy(
            src_hbm.at[pl.ds(base + i * chunk, chunk)], bufs[i % 2], sin[i % 2])

    def out_copy(i):
        return pltpu.make_async_copy(
            bufs[i % 2], out_hbm.at[pl.ds(base + i * chunk, chunk)], sout[i % 2])

    in_copy(0).start()
    for i in range(nchunks):
        in_copy(i).wait()
        out_copy(i).start()
        if i + 1 < nchunks:
            if i >= 1:
                out_copy(i - 1).wait()
            in_copy(i + 1).start()
    for i in range(max(0, nchunks - 2), nchunks):
        out_copy(i).wait()


def kernel(variable_features, constraint_features, edge_indices, reversed_edge_indices):
    n_var, d = variable_features.shape
    flat = variable_features.reshape(-1)
    per_worker = flat.shape[0] // _NW
    chunk = per_worker
    while chunk * 8 > 400_000:
        chunk //= 2
    mesh = plsc.VectorSubcoreMesh(core_axis_name="c", subcore_axis_name="s")
    out = pl.kernel(
        _sc_copy_body,
        out_type=jax.ShapeDtypeStruct(flat.shape, flat.dtype),
        mesh=mesh,
        scratch_types=[
            pltpu.VMEM((chunk,), jnp.float32),
            pltpu.VMEM((chunk,), jnp.float32),
            pltpu.SemaphoreType.DMA,
            pltpu.SemaphoreType.DMA,
            pltpu.SemaphoreType.DMA,
            pltpu.SemaphoreType.DMA,
        ],
    )(flat)
    return out.reshape(n_var, d)


# speedup vs baseline: 1.0527x; 1.0527x over previous
import jax
import jax.numpy as jnp
from jax import lax
from jax.experimental import pallas as pl
from jax.experimental.pallas import tpu as pltpu
from jax.experimental.pallas import tpu_sc as plsc

_INFO = plsc.get_sparse_core_info()
_NC = _INFO.num_cores
_NS = _INFO.num_subcores
_NW = _NC * _NS


def _sc_copy_body(src_hbm, out_hbm, buf):
    wid = lax.axis_index("s") * _NC + lax.axis_index("c")
    n = src_hbm.shape[0] // _NW
    chunk = buf.shape[0]
    base = wid * n
    for i in range(n // chunk):
        off = base + i * chunk
        pltpu.sync_copy(src_hbm.at[pl.ds(off, chunk)], buf)
        pltpu.sync_copy(buf, out_hbm.at[pl.ds(off, chunk)])


def kernel(variable_features, constraint_features, edge_indices, reversed_edge_indices):
    n_var, d = variable_features.shape
    flat = variable_features.reshape(-1)
    per_worker = flat.shape[0] // _NW
    chunk = per_worker
    while chunk * 4 > 400_000:
        chunk //= 2
    mesh = plsc.VectorSubcoreMesh(core_axis_name="c", subcore_axis_name="s")
    out = pl.kernel(
        _sc_copy_body,
        out_type=jax.ShapeDtypeStruct(flat.shape, flat.dtype),
        mesh=mesh,
        scratch_types=[pltpu.VMEM((chunk,), jnp.float32)],
    )(flat)
    return out.reshape(n_var, d)
